# baseline (speedup 1.0000x reference)
import jax
import jax.numpy as jnp
from jax.experimental import pallas as pl
from jax.experimental.pallas import tpu as pltpu

_TS = 64


def _router_body(x_ref, gt_ref, w_ref, b_ref, out_ref, sl_ref):
    j = pl.program_id(1)

    @pl.when(j == 0)
    def _():
        out_ref[0] = gt_ref[...]

    @pl.when(j > 0)
    def _():
        xb = x_ref[0]
        out_ref[0] = xb
        r = jnp.dot(xb, w_ref[...], preferred_element_type=jnp.float32)
        r = r + b_ref[...]
        lane = jax.lax.broadcasted_iota(jnp.int32, r.shape, 1)
        sl_ref[0] = jnp.where(lane == 0, jax.nn.sigmoid(r), r)


def kernel(x, global_tokens, anchor_w, anchor_b, seg_w, seg_b):
    b, s, h = x.shape
    g = global_tokens.shape[0]
    n = s // _TS

    w = jnp.concatenate([anchor_w, seg_w], axis=1)
    bias = jnp.stack([anchor_b[0], seg_b[0]]).reshape(1, 2)

    out, sl = pl.pallas_call(
        _router_body,
        grid=(b, n + 1),
        in_specs=[
            pl.BlockSpec((1, _TS, h), lambda i, j: (i, jnp.maximum(j - 1, 0), 0)),
            pl.BlockSpec((g, h), lambda i, j: (0, 0)),
            pl.BlockSpec((h, 2), lambda i, j: (0, 0)),
            pl.BlockSpec((1, 2), lambda i, j: (0, 0)),
        ],
        out_specs=[
            pl.BlockSpec((1, _TS, h), lambda i, j: (i, j, 0)),
            pl.BlockSpec((1, _TS, 2), lambda i, j: (i, jnp.maximum(j - 1, 0), 0)),
        ],
        out_shape=[
            jax.ShapeDtypeStruct((b, g + s, h), jnp.float32),
            jax.ShapeDtypeStruct((b, s, 2), jnp.float32),
        ],
        compiler_params=pltpu.CompilerParams(
            dimension_semantics=("parallel", "arbitrary"),
        ),
    )(x, global_tokens, w, bias)

    anchor_scores = sl[:, :, 0]
    segment_logits = sl[:, :, 1]
    mask_row = jnp.arange(g + s, dtype=jnp.int32) < g
    global_mask = jnp.broadcast_to(mask_row[None, :], (b, g + s))
    return (out, global_mask, anchor_scores, segment_logits)


# speedup vs baseline: 2.9318x; 2.9318x over previous
import jax
import jax.numpy as jnp
from jax.experimental import pallas as pl
from jax.experimental.pallas import tpu as pltpu

_TS = 1024
_G = 64


def _router_body(x_ref, gt_ref, w_ref, b_ref, sl_ref, out_ref, sem, gsem):
    bi = pl.program_id(0)
    i = pl.program_id(1)

    cp = pltpu.make_async_copy(
        x_ref.at[0],
        out_ref.at[bi, pl.ds(_G + i * _TS, _TS), :],
        sem,
    )
    cp.start()

    @pl.when(i == 0)
    def _():
        gcp = pltpu.make_async_copy(gt_ref, out_ref.at[bi, pl.ds(0, _G), :], gsem)
        gcp.start()
        gcp.wait()

    xb = x_ref[0]
    r = jnp.dot(xb, w_ref[...], preferred_element_type=jnp.float32)
    r = r + b_ref[...]
    lane = jax.lax.broadcasted_iota(jnp.int32, r.shape, 1)
    sl_ref[0] = jnp.where(lane == 0, jax.nn.sigmoid(r), r)

    cp.wait()


def kernel(x, global_tokens, anchor_w, anchor_b, seg_w, seg_b):
    b, s, h = x.shape
    g = global_tokens.shape[0]
    n = s // _TS

    w = jnp.concatenate([anchor_w, seg_w], axis=1)
    bias = jnp.stack([anchor_b[0], seg_b[0]]).reshape(1, 2)

    sl, out = pl.pallas_call(
        _router_body,
        grid=(b, n),
        in_specs=[
            pl.BlockSpec((1, _TS, h), lambda i, j: (i, j, 0)),
            pl.BlockSpec((g, h), lambda i, j: (0, 0)),
            pl.BlockSpec((h, 2), lambda i, j: (0, 0)),
            pl.BlockSpec((1, 2), lambda i, j: (0, 0)),
        ],
        out_specs=[
            pl.BlockSpec((1, _TS, 2), lambda i, j: (i, j, 0)),
            pl.BlockSpec(memory_space=pltpu.HBM),
        ],
        out_shape=[
            jax.ShapeDtypeStruct((b, s, 2), jnp.float32),
            jax.ShapeDtypeStruct((b, g + s, h), jnp.float32),
        ],
        scratch_shapes=[pltpu.SemaphoreType.DMA, pltpu.SemaphoreType.DMA],
    )(x, global_tokens, w, bias)

    anchor_scores = sl[:, :, 0]
    segment_logits = sl[:, :, 1]
    mask_row = jnp.arange(g + s, dtype=jnp.int32) < g
    global_mask = jnp.broadcast_to(mask_row[None, :], (b, g + s))
    return (out, global_mask, anchor_scores, segment_logits)


# speedup vs baseline: 3.1038x; 1.0587x over previous
import jax
import jax.numpy as jnp
from jax.experimental import pallas as pl
from jax.experimental.pallas import tpu as pltpu

_TS = 2048
_G = 64


def _router_body(x_ref, gt_ref, w_ref, b_ref, sl_ref, out_ref, sem, gsem):
    bi = pl.program_id(0)
    i = pl.program_id(1)

    cp = pltpu.make_async_copy(
        x_ref.at[0],
        out_ref.at[bi, pl.ds(_G + i * _TS, _TS), :],
        sem,
    )
    cp.start()

    @pl.when(i == 0)
    def _():
        gcp = pltpu.make_async_copy(gt_ref, out_ref.at[bi, pl.ds(0, _G), :], gsem)
        gcp.start()
        gcp.wait()

    xb = x_ref[0]
    r = jnp.dot(xb, w_ref[...], preferred_element_type=jnp.float32)
    r = r + b_ref[...]
    lane = jax.lax.broadcasted_iota(jnp.int32, r.shape, 1)
    sl_ref[0] = jnp.where(lane == 0, jax.nn.sigmoid(r), r)

    cp.wait()


def kernel(x, global_tokens, anchor_w, anchor_b, seg_w, seg_b):
    b, s, h = x.shape
    g = global_tokens.shape[0]
    n = s // _TS

    w = jnp.concatenate([anchor_w, seg_w], axis=1)
    bias = jnp.stack([anchor_b[0], seg_b[0]]).reshape(1, 2)

    sl, out = pl.pallas_call(
        _router_body,
        grid=(b, n),
        in_specs=[
            pl.BlockSpec((1, _TS, h), lambda i, j: (i, j, 0)),
            pl.BlockSpec((g, h), lambda i, j: (0, 0)),
            pl.BlockSpec((h, 2), lambda i, j: (0, 0)),
            pl.BlockSpec((1, 2), lambda i, j: (0, 0)),
        ],
        out_specs=[
            pl.BlockSpec((1, _TS, 2), lambda i, j: (i, j, 0)),
            pl.BlockSpec(memory_space=pltpu.HBM),
        ],
        out_shape=[
            jax.ShapeDtypeStruct((b, s, 2), jnp.float32),
            jax.ShapeDtypeStruct((b, g + s, h), jnp.float32),
        ],
        scratch_shapes=[pltpu.SemaphoreType.DMA, pltpu.SemaphoreType.DMA],
    )(x, global_tokens, w, bias)

    anchor_scores = sl[:, :, 0]
    segment_logits = sl[:, :, 1]
    mask_row = jnp.arange(g + s, dtype=jnp.int32) < g
    global_mask = jnp.broadcast_to(mask_row[None, :], (b, g + s))
    return (out, global_mask, anchor_scores, segment_logits)
